# baseline (speedup 1.0000x reference)
import functools

import jax
import jax.numpy as jnp
from jax import lax
from jax.experimental import pallas as pl
from jax.experimental.pallas import tpu as pltpu
from jax.experimental.pallas import tpu_sc as plsc

K = 128
T = 512
LANES = 16


def _lookup_body(x_hbm, et_hbm, mul_hbm, bias_hbm, out_hbm,
                 x_v, et_v, xx_v, mul_v, bias_v, sem_x, sem_et, sem_t, sem_o,
                 *, chunk, num_cores):
    wid = lax.axis_index("s") * num_cores + lax.axis_index("c")
    base = wid * chunk
    half = chunk // 2
    cp_x0 = pltpu.async_copy(x_hbm.at[pl.ds(base, half)],
                             x_v.at[pl.ds(0, half)], sem_x)
    cp_et0 = pltpu.async_copy(et_hbm.at[pl.ds(base, half)],
                              et_v.at[pl.ds(0, half)], sem_et)
    cp_m = pltpu.async_copy(mul_hbm, mul_v, sem_t)
    cp_b = pltpu.async_copy(bias_hbm, bias_v, sem_t)
    cp_x1 = pltpu.async_copy(x_hbm.at[pl.ds(base + half, half)],
                             x_v.at[pl.ds(half, half)], sem_x)
    cp_et1 = pltpu.async_copy(et_hbm.at[pl.ds(base + half, half)],
                              et_v.at[pl.ds(half, half)], sem_et)
    cp_m.wait()
    cp_b.wait()
    cp_x0.wait()
    cp_et0.wait()

    @plsc.parallel_loop(0, half, LANES, unroll=8)
    def body0(i):
        sl = pl.ds(i, LANES)
        idx = et_v[sl]
        m = plsc.load_gather(mul_v, [idx])
        b = plsc.load_gather(bias_v, [idx])
        xx_v[sl] = m * x_v[sl] + b

    cp_o0 = pltpu.async_copy(xx_v.at[pl.ds(0, half)],
                             out_hbm.at[pl.ds(base, half)], sem_o)
    cp_x1.wait()
    cp_et1.wait()

    @plsc.parallel_loop(half, chunk, LANES, unroll=8)
    def body1(i):
        sl = pl.ds(i, LANES)
        idx = et_v[sl]
        m = plsc.load_gather(mul_v, [idx])
        b = plsc.load_gather(bias_v, [idx])
        xx_v[sl] = m * x_v[sl] + b

    cp_o0.wait()
    pltpu.sync_copy(xx_v.at[pl.ds(half, half)],
                    out_hbm.at[pl.ds(base + half, half)])


def _sc_lookup(x, et, mul_w, bias_w):
    E = x.shape[0]
    try:
        info = plsc.get_sparse_core_info()
        nc, ns = info.num_cores, info.num_subcores
    except ValueError:
        nc, ns = 2, 16
    nw = nc * ns
    chunk = E // nw
    assert E % (nw * LANES) == 0 and chunk % 8 == 0
    mesh = plsc.VectorSubcoreMesh(core_axis_name="c", subcore_axis_name="s",
                                  num_cores=nc, num_subcores=ns)
    fn = functools.partial(
        pl.kernel,
        out_type=jax.ShapeDtypeStruct((E,), jnp.float32),
        mesh=mesh,
        name="sc_affine_lookup",
        compiler_params=pltpu.CompilerParams(needs_layout_passes=False),
        scratch_types=[
            pltpu.VMEM((chunk,), jnp.float32),
            pltpu.VMEM((chunk,), jnp.int32),
            pltpu.VMEM((chunk,), jnp.float32),
            pltpu.VMEM((T,), jnp.float32),
            pltpu.VMEM((T,), jnp.float32),
            pltpu.SemaphoreType.DMA,
            pltpu.SemaphoreType.DMA,
            pltpu.SemaphoreType.DMA,
            pltpu.SemaphoreType.DMA,
        ],
    )(functools.partial(_lookup_body, chunk=chunk, num_cores=info.num_cores))
    return fn(x, et, mul_w, bias_w)


def _rbf_body(xx_ref, means_ref, temps_ref, out_ref, *, block):
    m = means_ref[:]
    t = jnp.abs(temps_ref[:])
    log2e = jnp.float32(1.4426950408889634)
    tl = t * log2e
    c = -tl
    b = 2.0 * tl * m
    a = -tl * m * m
    xx = xx_ref[:]
    xxb = jnp.broadcast_to(xx[:, None], (block, K))
    z = xxb * (c[None, :] * xxb + b[None, :]) + a[None, :]
    out_ref[:, :] = jnp.exp2(z)


def kernel(x, edge_types, means, temps, mul_weight, bias_weight):
    E = x.shape[0]
    xx = _sc_lookup(x, edge_types.astype(jnp.int32),
                    mul_weight.reshape(-1), bias_weight.reshape(-1))
    B = 25600
    assert E % B == 0
    out = pl.pallas_call(
        functools.partial(_rbf_body, block=B),
        grid=(E // B,),
        in_specs=[
            pl.BlockSpec((B,), lambda i: (i,)),
            pl.BlockSpec((K,), lambda i: (0,)),
            pl.BlockSpec((K,), lambda i: (0,)),
        ],
        out_specs=pl.BlockSpec((B, K), lambda i: (i, 0)),
        out_shape=jax.ShapeDtypeStruct((E, K), jnp.float32),
        compiler_params=pltpu.CompilerParams(
            dimension_semantics=("arbitrary",),
            vmem_limit_bytes=134217728),
    )(xx, means, temps)
    return out.astype(means.dtype)


# speedup vs baseline: 1.0213x; 1.0213x over previous
import functools

import jax
import jax.numpy as jnp
from jax import lax
from jax.experimental import pallas as pl
from jax.experimental.pallas import tpu as pltpu
from jax.experimental.pallas import tpu_sc as plsc

K = 128
T = 512
LANES = 16


def _lookup_body(x_hbm, et_hbm, mul_hbm, bias_hbm, out_hbm,
                 x_v, et_v, xx_v, mul_v, bias_v, sem_x, sem_et, sem_t, sem_o,
                 *, chunk, num_cores):
    wid = lax.axis_index("s") * num_cores + lax.axis_index("c")
    base = wid * chunk
    half = chunk // 2
    cp_x0 = pltpu.async_copy(x_hbm.at[pl.ds(base, half)],
                             x_v.at[pl.ds(0, half)], sem_x)
    cp_et0 = pltpu.async_copy(et_hbm.at[pl.ds(base, half)],
                              et_v.at[pl.ds(0, half)], sem_et)
    cp_m = pltpu.async_copy(mul_hbm, mul_v, sem_t)
    cp_b = pltpu.async_copy(bias_hbm, bias_v, sem_t)
    cp_x1 = pltpu.async_copy(x_hbm.at[pl.ds(base + half, half)],
                             x_v.at[pl.ds(half, half)], sem_x)
    cp_et1 = pltpu.async_copy(et_hbm.at[pl.ds(base + half, half)],
                              et_v.at[pl.ds(half, half)], sem_et)
    cp_m.wait()
    cp_b.wait()
    cp_x0.wait()
    cp_et0.wait()

    @plsc.parallel_loop(0, half, LANES, unroll=8)
    def body0(i):
        sl = pl.ds(i, LANES)
        idx = et_v[sl]
        m = plsc.load_gather(mul_v, [idx])
        b = plsc.load_gather(bias_v, [idx])
        xx_v[sl] = m * x_v[sl] + b

    cp_o0 = pltpu.async_copy(xx_v.at[pl.ds(0, half)],
                             out_hbm.at[pl.ds(base, half)], sem_o)
    cp_x1.wait()
    cp_et1.wait()

    @plsc.parallel_loop(half, chunk, LANES, unroll=8)
    def body1(i):
        sl = pl.ds(i, LANES)
        idx = et_v[sl]
        m = plsc.load_gather(mul_v, [idx])
        b = plsc.load_gather(bias_v, [idx])
        xx_v[sl] = m * x_v[sl] + b

    cp_o0.wait()
    pltpu.sync_copy(xx_v.at[pl.ds(half, half)],
                    out_hbm.at[pl.ds(base + half, half)])


def _sc_lookup(x, et, mul_w, bias_w):
    E = x.shape[0]
    try:
        info = plsc.get_sparse_core_info()
        nc, ns = info.num_cores, info.num_subcores
    except ValueError:
        nc, ns = 2, 16
    nw = nc * ns
    chunk = E // nw
    assert E % (nw * LANES) == 0 and chunk % 8 == 0
    mesh = plsc.VectorSubcoreMesh(core_axis_name="c", subcore_axis_name="s",
                                  num_cores=nc, num_subcores=ns)
    fn = functools.partial(
        pl.kernel,
        out_type=jax.ShapeDtypeStruct((E,), jnp.float32),
        mesh=mesh,
        name="sc_affine_lookup",
        compiler_params=pltpu.CompilerParams(needs_layout_passes=False),
        scratch_types=[
            pltpu.VMEM((chunk,), jnp.float32),
            pltpu.VMEM((chunk,), jnp.int32),
            pltpu.VMEM((chunk,), jnp.float32),
            pltpu.VMEM((T,), jnp.float32),
            pltpu.VMEM((T,), jnp.float32),
            pltpu.SemaphoreType.DMA,
            pltpu.SemaphoreType.DMA,
            pltpu.SemaphoreType.DMA,
            pltpu.SemaphoreType.DMA,
        ],
    )(functools.partial(_lookup_body, chunk=chunk, num_cores=nc))
    return fn(x, et, mul_w, bias_w)


def _rbf_body(xx_ref, means_ref, temps_ref, out_ref, *, block):
    m = means_ref[:]
    t = jnp.abs(temps_ref[:])
    log2e = jnp.float32(1.4426950408889634)
    tl = t * log2e
    c = -tl
    b = 2.0 * tl * m
    a = -tl * m * m
    xx = xx_ref[:]
    xxb = jnp.broadcast_to(xx[:, None], (block, K))
    z = xxb * (c[None, :] * xxb + b[None, :]) + a[None, :]
    out_ref[:, :] = jnp.exp2(z)


def kernel(x, edge_types, means, temps, mul_weight, bias_weight):
    E = x.shape[0]
    xx = _sc_lookup(x, edge_types.astype(jnp.int32),
                    mul_weight.reshape(-1), bias_weight.reshape(-1))
    B = 25600
    assert E % B == 0
    out = pl.pallas_call(
        functools.partial(_rbf_body, block=B),
        grid=(E // B,),
        in_specs=[
            pl.BlockSpec((B,), lambda i: (i,)),
            pl.BlockSpec((K,), lambda i: (0,)),
            pl.BlockSpec((K,), lambda i: (0,)),
        ],
        out_specs=pl.BlockSpec((B, K), lambda i: (i, 0)),
        out_shape=jax.ShapeDtypeStruct((E, K), jnp.float32),
        compiler_params=pltpu.CompilerParams(
            dimension_semantics=("arbitrary",)),
    )(xx, means, temps)
    return out.astype(means.dtype)
